# baseline (speedup 1.0000x reference)
import numpy as np
import jax
import jax.numpy as jnp
from jax import lax
from jax.experimental import pallas as pl
from jax.experimental.pallas import tpu as pltpu

ROWS = 64
VOCAB = 1_000_000
CHUNK = 1024
NCHUNK = (VOCAB + CHUNK - 1) // CHUNK

_ROT_A = (13, 15, 26, 6)
_ROT_B = (17, 29, 16, 24)


def _np_threefry2x32(k0, k1, x0, x1):
    m = np.uint32(0xFFFFFFFF)
    ks = [np.uint32(k0), np.uint32(k1),
          np.uint32(k0) ^ np.uint32(k1) ^ np.uint32(0x1BD11BDA)]
    x0 = np.uint32(x0 + ks[0]) & m
    x1 = np.uint32(x1 + ks[1]) & m
    for i, rots in enumerate([_ROT_A, _ROT_B, _ROT_A, _ROT_B, _ROT_A]):
        for r in rots:
            x0 = np.uint32((int(x0) + int(x1)) & 0xFFFFFFFF)
            x1 = np.uint32(((int(x1) << r) | (int(x1) >> (32 - r))) & 0xFFFFFFFF)
            x1 = x0 ^ x1
        j = i + 1
        x0 = np.uint32((int(x0) + int(ks[j % 3])) & 0xFFFFFFFF)
        x1 = np.uint32((int(x1) + int(ks[(j + 1) % 3]) + j) & 0xFFFFFFFF)
    return x0, x1


_K0, _K1 = _np_threefry2x32(0, 0, 0, 1)
_K0, _K1 = np.uint32(_K0), np.uint32(_K1)
_KS = (_K0, _K1, np.uint32(_K0 ^ _K1 ^ np.uint32(0x1BD11BDA)))


def _gumbel(flat_u32):
    x0 = jnp.full_like(flat_u32, _K0)
    x1 = flat_u32 + _K1
    for i, rots in enumerate([_ROT_A, _ROT_B, _ROT_A, _ROT_B, _ROT_A]):
        for r in rots:
            x0 = x0 + x1
            x1 = (x1 << np.uint32(r)) | (x1 >> np.uint32(32 - r))
            x1 = x0 ^ x1
        j = i + 1
        x0 = x0 + _KS[j % 3]
        x1 = x1 + _KS[(j + 1) % 3] + np.uint32(j)
    bits = x0 ^ x1
    fb = (bits >> np.uint32(9)) | np.uint32(0x3F800000)
    u = lax.bitcast_convert_type(fb, jnp.float32) - jnp.float32(1.0)
    return -jnp.log(-jnp.log(u))


def _body(x_ref, out_ref, bv_ref, bi_ref):
    step = pl.program_id(0)
    base = step * CHUNK
    col = lax.broadcasted_iota(jnp.int32, (ROWS, CHUNK), 1) + base
    row = lax.broadcasted_iota(jnp.int32, (ROWS, CHUNK), 0)
    flat = (row * VOCAB + col).astype(jnp.uint32)

    y = x_ref[...] + _gumbel(flat)
    y = jnp.where(col < VOCAB, y, -jnp.inf)

    m = jnp.max(y, axis=1, keepdims=True)
    idx = jnp.min(jnp.where(y == m, col, jnp.int32(2**31 - 1)),
                  axis=1, keepdims=True)

    @pl.when(step == 0)
    def _():
        bv_ref[...] = m
        bi_ref[...] = idx

    @pl.when(step > 0)
    def _():
        better = m > bv_ref[...]
        bv_ref[...] = jnp.where(better, m, bv_ref[...])
        bi_ref[...] = jnp.where(better, idx, bi_ref[...])

    @pl.when(step == NCHUNK - 1)
    def _():
        out_ref[...] = bi_ref[...]


def kernel(x):
    out = pl.pallas_call(
        _body,
        grid=(NCHUNK,),
        in_specs=[pl.BlockSpec((ROWS, CHUNK), lambda i: (0, i))],
        out_specs=pl.BlockSpec((ROWS, 1), lambda i: (0, 0)),
        out_shape=jax.ShapeDtypeStruct((ROWS, 1), jnp.int32),
        scratch_shapes=[
            pltpu.VMEM((ROWS, 1), jnp.float32),
            pltpu.VMEM((ROWS, 1), jnp.int32),
        ],
    )(x)
    return out.reshape(ROWS)


# speedup vs baseline: 1.3103x; 1.3103x over previous
import numpy as np
import jax
import jax.numpy as jnp
from jax import lax
from jax.experimental import pallas as pl
from jax.experimental.pallas import tpu as pltpu
from jax.experimental.pallas import tpu_sc as plsc
import functools

ROWS = 64
VOCAB = 1_000_000

CHUNK = 2048
SC_BLK = 2048
SC_NBLK = 96
SC_COLS = SC_NBLK * SC_BLK
TAILBLK = VOCAB // CHUNK
TC_COLS = TAILBLK * CHUNK - SC_COLS
NT1 = TC_COLS // CHUNK
NT = NT1 + 1

_ROT_A = (13, 15, 26, 6)
_ROT_B = (17, 29, 16, 24)


def _np_threefry2x32(k0, k1, x0, x1):
    ks = [k0, k1, k0 ^ k1 ^ 0x1BD11BDA]
    x0 = (x0 + ks[0]) & 0xFFFFFFFF
    x1 = (x1 + ks[1]) & 0xFFFFFFFF
    for i, rots in enumerate([_ROT_A, _ROT_B, _ROT_A, _ROT_B, _ROT_A]):
        for r in rots:
            x0 = (x0 + x1) & 0xFFFFFFFF
            x1 = ((x1 << r) | (x1 >> (32 - r))) & 0xFFFFFFFF
            x1 = x0 ^ x1
        j = i + 1
        x0 = (x0 + ks[j % 3]) & 0xFFFFFFFF
        x1 = (x1 + ks[(j + 1) % 3] + j) & 0xFFFFFFFF
    return x0, x1


_K0, _K1 = (np.uint32(v) for v in _np_threefry2x32(0, 0, 0, 1))
_KS = (_K0, _K1, np.uint32(_K0 ^ _K1 ^ np.uint32(0x1BD11BDA)))


def _threefry_bits(flat_u32):
    x0 = jnp.full_like(flat_u32, _K0)
    x1 = flat_u32 + _K1
    for i, rots in enumerate([_ROT_A, _ROT_B, _ROT_A, _ROT_B, _ROT_A]):
        for r in rots:
            x0 = x0 + x1
            x1 = (x1 << np.uint32(r)) | (x1 >> np.uint32(32 - r))
            x1 = x0 ^ x1
        j = i + 1
        x0 = x0 + _KS[j % 3]
        x1 = x1 + _KS[(j + 1) % 3] + np.uint32(j)
    return x0 ^ x1


def _bits_to_u(bits):
    fb = (bits >> np.uint32(9)) | np.uint32(0x3F800000)
    return lax.bitcast_convert_type(fb, jnp.float32) - jnp.float32(1.0)


def _tc_body(x_ref, val_ref, idx_ref):
    step = pl.program_id(0)
    base = jnp.where(step < NT1, step, TAILBLK) * CHUNK
    col = lax.broadcasted_iota(jnp.int32, (ROWS, CHUNK), 1) + base
    row = lax.broadcasted_iota(jnp.int32, (ROWS, CHUNK), 0)
    flat = (row * VOCAB + col).astype(jnp.uint32)

    u = _bits_to_u(_threefry_bits(flat))
    y = x_ref[...] + (-jnp.log(-jnp.log(u)))
    y = jnp.where(col < VOCAB, y, -jnp.inf)

    m = jnp.max(y, axis=1, keepdims=True)
    idx = jnp.min(jnp.where(y == m, col, jnp.int32(2**31 - 1)),
                  axis=1, keepdims=True)

    @pl.when(step == 0)
    def _():
        val_ref[...] = m
        idx_ref[...] = idx

    @pl.when(step > 0)
    def _():
        better = m > val_ref[...]
        val_ref[...] = jnp.where(better, m, val_ref[...])
        idx_ref[...] = jnp.where(better, idx, idx_ref[...])


_LN2 = np.float32(0.6931471805599453)
_SQRT2 = np.float32(1.4142135623730951)
_QC = (np.float32(-0.1043442), np.float32(0.16080349), np.float32(-0.17102107),
       np.float32(0.19918212), np.float32(-0.24983448), np.float32(0.33334355),
       np.float32(-0.50000086))


def _ln16(v):
    bits = lax.bitcast_convert_type(v, jnp.uint32)
    e = (bits >> np.uint32(23)).astype(jnp.int32) - 127
    m = lax.bitcast_convert_type(
        (bits & np.uint32(0x7FFFFF)) | np.uint32(0x3F800000), jnp.float32)
    big = m >= _SQRT2
    e = jnp.where(big, e + 1, e).astype(jnp.float32)
    m = jnp.where(big, m * np.float32(0.5), m)
    t = m - np.float32(1.0)
    q = _QC[0]
    for c in _QC[1:]:
        q = q * t + c
    return e * _LN2 + (t + t * t * q)


@functools.partial(
    pl.kernel,
    mesh=plsc.VectorSubcoreMesh(core_axis_name="c", subcore_axis_name="s"),
    compiler_params=pltpu.CompilerParams(needs_layout_passes=False),
    out_type=[
        jax.ShapeDtypeStruct((ROWS, 16), jnp.float32),
        jax.ShapeDtypeStruct((ROWS, 16), jnp.int32),
    ],
    scratch_types=[
        pltpu.VMEM((SC_BLK,), jnp.float32),
        pltpu.VMEM((16,), jnp.float32),
        pltpu.VMEM((16,), jnp.int32),
    ],
)
def _sc_kernel(x_hbm, vals_hbm, idxs_hbm, xb, resv, resi):
    wid = lax.axis_index("s") * 2 + lax.axis_index("c")
    lanes = lax.iota(jnp.int32, 16)

    def scan_row(row):
        rowbase = row * VOCAB

        def inner(j, carry, col0):
            bestv, besti = carry
            xv = xb[pl.ds(j * 16, 16)]
            col = col0 + j * 16 + lanes
            flat = (rowbase + col).astype(jnp.uint32)
            u = _bits_to_u(_threefry_bits(flat))
            g = -_ln16(-_ln16(u))
            y = xv + g
            pred = y > bestv
            return jnp.where(pred, y, bestv), jnp.where(pred, col, besti)

        def block(b, carry):
            col0 = TC_COLS + b * SC_BLK
            pltpu.sync_copy(x_hbm.at[row, pl.ds(col0, SC_BLK)], xb)
            return lax.fori_loop(0, SC_BLK // 16,
                                 lambda j, c: inner(j, c, col0), carry)

        carry = (jnp.full((16,), -jnp.inf, jnp.float32),
                 jnp.zeros((16,), jnp.int32))
        bestv, besti = lax.fori_loop(0, SC_NBLK, block, carry)
        resv[...] = bestv
        resi[...] = besti
        pltpu.sync_copy(resv, vals_hbm.at[row])
        pltpu.sync_copy(resi, idxs_hbm.at[row])

    scan_row(wid * 2)
    scan_row(wid * 2 + 1)


def _merge_body(tcv_ref, tci_ref, scv_ref, sci_ref, out_ref):
    scv = scv_ref[...]
    m = jnp.max(scv, axis=1, keepdims=True)
    si = jnp.min(jnp.where(scv == m, sci_ref[...], jnp.int32(2**31 - 1)),
                 axis=1, keepdims=True)
    out_ref[...] = jnp.where(m > tcv_ref[...], si, tci_ref[...])


def kernel(x):
    tcv, tci = pl.pallas_call(
        _tc_body,
        grid=(NT,),
        in_specs=[pl.BlockSpec((ROWS, CHUNK),
                               lambda i: (0, jnp.where(i < NT1, i, TAILBLK)))],
        out_specs=[pl.BlockSpec((ROWS, 1), lambda i: (0, 0)),
                   pl.BlockSpec((ROWS, 1), lambda i: (0, 0))],
        out_shape=[jax.ShapeDtypeStruct((ROWS, 1), jnp.float32),
                   jax.ShapeDtypeStruct((ROWS, 1), jnp.int32)],
    )(x)
    scv, sci = _sc_kernel(x)
    out = pl.pallas_call(
        _merge_body,
        out_shape=jax.ShapeDtypeStruct((ROWS, 1), jnp.int32),
    )(tcv, tci, scv, sci)
    return out.reshape(ROWS)


# speedup vs baseline: 1.3103x; 1.0000x over previous
import numpy as np
import jax
import jax.numpy as jnp
from jax import lax
from jax.experimental import pallas as pl
from jax.experimental.pallas import tpu as pltpu
from jax.experimental.pallas import tpu_sc as plsc
import functools

ROWS = 64
VOCAB = 1_000_000

CHUNK = 2048
SC_BLK = 16384
SC_NBLK = 12
SC_COLS = SC_NBLK * SC_BLK
TAILBLK = VOCAB // CHUNK
TC_COLS = TAILBLK * CHUNK - SC_COLS
NT1 = TC_COLS // CHUNK
NT = NT1 + 1

_ROT_A = (13, 15, 26, 6)
_ROT_B = (17, 29, 16, 24)


def _np_threefry2x32(k0, k1, x0, x1):
    ks = [k0, k1, k0 ^ k1 ^ 0x1BD11BDA]
    x0 = (x0 + ks[0]) & 0xFFFFFFFF
    x1 = (x1 + ks[1]) & 0xFFFFFFFF
    for i, rots in enumerate([_ROT_A, _ROT_B, _ROT_A, _ROT_B, _ROT_A]):
        for r in rots:
            x0 = (x0 + x1) & 0xFFFFFFFF
            x1 = ((x1 << r) | (x1 >> (32 - r))) & 0xFFFFFFFF
            x1 = x0 ^ x1
        j = i + 1
        x0 = (x0 + ks[j % 3]) & 0xFFFFFFFF
        x1 = (x1 + ks[(j + 1) % 3] + j) & 0xFFFFFFFF
    return x0, x1


_K0, _K1 = (np.uint32(v) for v in _np_threefry2x32(0, 0, 0, 1))
_KS = (_K0, _K1, np.uint32(_K0 ^ _K1 ^ np.uint32(0x1BD11BDA)))


def _threefry_bits(flat_u32):
    x0 = jnp.full_like(flat_u32, _K0)
    x1 = flat_u32 + _K1
    for i, rots in enumerate([_ROT_A, _ROT_B, _ROT_A, _ROT_B, _ROT_A]):
        for r in rots:
            x0 = x0 + x1
            x1 = (x1 << np.uint32(r)) | (x1 >> np.uint32(32 - r))
            x1 = x0 ^ x1
        j = i + 1
        x0 = x0 + _KS[j % 3]
        x1 = x1 + _KS[(j + 1) % 3] + np.uint32(j)
    return x0 ^ x1


def _bits_to_u(bits):
    fb = (bits >> np.uint32(9)) | np.uint32(0x3F800000)
    return lax.bitcast_convert_type(fb, jnp.float32) - jnp.float32(1.0)


def _tc_body(x_ref, val_ref, idx_ref):
    step = pl.program_id(0)
    base = jnp.where(step < NT1, step, TAILBLK) * CHUNK
    col = lax.broadcasted_iota(jnp.int32, (ROWS, CHUNK), 1) + base
    row = lax.broadcasted_iota(jnp.int32, (ROWS, CHUNK), 0)
    flat = (row * VOCAB + col).astype(jnp.uint32)

    u = _bits_to_u(_threefry_bits(flat))
    y = x_ref[...] + (-jnp.log(-jnp.log(u)))
    y = jnp.where(col < VOCAB, y, -jnp.inf)

    m = jnp.max(y, axis=1, keepdims=True)
    idx = jnp.min(jnp.where(y == m, col, jnp.int32(2**31 - 1)),
                  axis=1, keepdims=True)

    @pl.when(step == 0)
    def _():
        val_ref[...] = m
        idx_ref[...] = idx

    @pl.when(step > 0)
    def _():
        better = m > val_ref[...]
        val_ref[...] = jnp.where(better, m, val_ref[...])
        idx_ref[...] = jnp.where(better, idx, idx_ref[...])


_LN2 = np.float32(0.6931471805599453)
_SQRT2 = np.float32(1.4142135623730951)
_QC = (np.float32(-0.1043442), np.float32(0.16080349), np.float32(-0.17102107),
       np.float32(0.19918212), np.float32(-0.24983448), np.float32(0.33334355),
       np.float32(-0.50000086))


def _ln16(v):
    bits = lax.bitcast_convert_type(v, jnp.uint32)
    e = (bits >> np.uint32(23)).astype(jnp.int32) - 127
    m = lax.bitcast_convert_type(
        (bits & np.uint32(0x7FFFFF)) | np.uint32(0x3F800000), jnp.float32)
    big = m >= _SQRT2
    e = jnp.where(big, e + 1, e).astype(jnp.float32)
    m = jnp.where(big, m * np.float32(0.5), m)
    t = m - np.float32(1.0)
    q = _QC[0]
    for c in _QC[1:]:
        q = q * t + c
    return e * _LN2 + (t + t * t * q)


@functools.partial(
    pl.kernel,
    mesh=plsc.VectorSubcoreMesh(core_axis_name="c", subcore_axis_name="s"),
    compiler_params=pltpu.CompilerParams(needs_layout_passes=False),
    out_type=[
        jax.ShapeDtypeStruct((ROWS, 16), jnp.float32),
        jax.ShapeDtypeStruct((ROWS, 16), jnp.int32),
    ],
    scratch_types=[
        pltpu.VMEM((SC_BLK,), jnp.float32),
        pltpu.VMEM((16,), jnp.float32),
        pltpu.VMEM((16,), jnp.int32),
    ],
)
def _sc_kernel(x_hbm, vals_hbm, idxs_hbm, xb, resv, resi):
    wid = lax.axis_index("s") * 2 + lax.axis_index("c")
    lanes = lax.iota(jnp.int32, 16)

    def scan_row(row):
        rowbase = row * VOCAB

        def inner(j, carry, col0):
            bestv, besti = carry
            xv = xb[pl.ds(j * 16, 16)]
            col = col0 + j * 16 + lanes
            flat = (rowbase + col).astype(jnp.uint32)
            u = _bits_to_u(_threefry_bits(flat))
            g = -_ln16(-_ln16(u))
            y = xv + g
            pred = y > bestv
            return jnp.where(pred, y, bestv), jnp.where(pred, col, besti)

        def block(b, carry):
            col0 = TC_COLS + b * SC_BLK
            pltpu.sync_copy(x_hbm.at[row, pl.ds(col0, SC_BLK)], xb)
            return lax.fori_loop(0, SC_BLK // 16,
                                 lambda j, c: inner(j, c, col0), carry)

        carry = (jnp.full((16,), -jnp.inf, jnp.float32),
                 jnp.zeros((16,), jnp.int32))
        bestv, besti = lax.fori_loop(0, SC_NBLK, block, carry)
        resv[...] = bestv
        resi[...] = besti
        pltpu.sync_copy(resv, vals_hbm.at[row])
        pltpu.sync_copy(resi, idxs_hbm.at[row])

    scan_row(wid * 2)
    scan_row(wid * 2 + 1)


def _merge_body(tcv_ref, tci_ref, scv_ref, sci_ref, out_ref):
    scv = scv_ref[...]
    m = jnp.max(scv, axis=1, keepdims=True)
    si = jnp.min(jnp.where(scv == m, sci_ref[...], jnp.int32(2**31 - 1)),
                 axis=1, keepdims=True)
    out_ref[...] = jnp.where(m > tcv_ref[...], si, tci_ref[...])


def kernel(x):
    tcv, tci = pl.pallas_call(
        _tc_body,
        grid=(NT,),
        in_specs=[pl.BlockSpec((ROWS, CHUNK),
                               lambda i: (0, jnp.where(i < NT1, i, TAILBLK)))],
        out_specs=[pl.BlockSpec((ROWS, 1), lambda i: (0, 0)),
                   pl.BlockSpec((ROWS, 1), lambda i: (0, 0))],
        out_shape=[jax.ShapeDtypeStruct((ROWS, 1), jnp.float32),
                   jax.ShapeDtypeStruct((ROWS, 1), jnp.int32)],
    )(x)
    scv, sci = _sc_kernel(x)
    out = pl.pallas_call(
        _merge_body,
        out_shape=jax.ShapeDtypeStruct((ROWS, 1), jnp.int32),
    )(tcv, tci, scv, sci)
    return out.reshape(ROWS)


# speedup vs baseline: 1.3234x; 1.0100x over previous
import numpy as np
import jax
import jax.numpy as jnp
from jax import lax
from jax.experimental import pallas as pl
from jax.experimental.pallas import tpu as pltpu
from jax.experimental.pallas import tpu_sc as plsc
import functools

ROWS = 64
VOCAB = 1_000_000

CHUNK = 2048
SC_BLK = 8192
SC_NBLK = 25
SC_COLS = SC_NBLK * SC_BLK
TAILBLK = VOCAB // CHUNK
TC_COLS = TAILBLK * CHUNK - SC_COLS
NT1 = TC_COLS // CHUNK
NT = NT1 + 1

_ROT_A = (13, 15, 26, 6)
_ROT_B = (17, 29, 16, 24)


def _np_threefry2x32(k0, k1, x0, x1):
    ks = [k0, k1, k0 ^ k1 ^ 0x1BD11BDA]
    x0 = (x0 + ks[0]) & 0xFFFFFFFF
    x1 = (x1 + ks[1]) & 0xFFFFFFFF
    for i, rots in enumerate([_ROT_A, _ROT_B, _ROT_A, _ROT_B, _ROT_A]):
        for r in rots:
            x0 = (x0 + x1) & 0xFFFFFFFF
            x1 = ((x1 << r) | (x1 >> (32 - r))) & 0xFFFFFFFF
            x1 = x0 ^ x1
        j = i + 1
        x0 = (x0 + ks[j % 3]) & 0xFFFFFFFF
        x1 = (x1 + ks[(j + 1) % 3] + j) & 0xFFFFFFFF
    return x0, x1


_K0, _K1 = (np.uint32(v) for v in _np_threefry2x32(0, 0, 0, 1))
_KS = (_K0, _K1, np.uint32(_K0 ^ _K1 ^ np.uint32(0x1BD11BDA)))


def _threefry_bits(flat_u32):
    x0 = jnp.full_like(flat_u32, _K0)
    x1 = flat_u32 + _K1
    for i, rots in enumerate([_ROT_A, _ROT_B, _ROT_A, _ROT_B, _ROT_A]):
        for r in rots:
            x0 = x0 + x1
            x1 = (x1 << np.uint32(r)) | (x1 >> np.uint32(32 - r))
            x1 = x0 ^ x1
        j = i + 1
        x0 = x0 + _KS[j % 3]
        x1 = x1 + _KS[(j + 1) % 3] + np.uint32(j)
    return x0 ^ x1


def _bits_to_u(bits):
    fb = (bits >> np.uint32(9)) | np.uint32(0x3F800000)
    return lax.bitcast_convert_type(fb, jnp.float32) - jnp.float32(1.0)


def _tc_body(x_ref, val_ref, idx_ref):
    step = pl.program_id(0)
    base = jnp.where(step < NT1, step, TAILBLK) * CHUNK
    col = lax.broadcasted_iota(jnp.int32, (ROWS, CHUNK), 1) + base
    row = lax.broadcasted_iota(jnp.int32, (ROWS, CHUNK), 0)
    flat = (row * VOCAB + col).astype(jnp.uint32)

    u = _bits_to_u(_threefry_bits(flat))
    y = x_ref[...] + (-jnp.log(-jnp.log(u)))
    y = jnp.where(col < VOCAB, y, -jnp.inf)

    m = jnp.max(y, axis=1, keepdims=True)
    idx = jnp.min(jnp.where(y == m, col, jnp.int32(2**31 - 1)),
                  axis=1, keepdims=True)

    @pl.when(step == 0)
    def _():
        val_ref[...] = m
        idx_ref[...] = idx

    @pl.when(step > 0)
    def _():
        better = m > val_ref[...]
        val_ref[...] = jnp.where(better, m, val_ref[...])
        idx_ref[...] = jnp.where(better, idx, idx_ref[...])


_LN2 = np.float32(0.6931471805599453)
_SQRT2 = np.float32(1.4142135623730951)
_QC = (np.float32(-0.1043442), np.float32(0.16080349), np.float32(-0.17102107),
       np.float32(0.19918212), np.float32(-0.24983448), np.float32(0.33334355),
       np.float32(-0.50000086))


def _ln16(v):
    bits = lax.bitcast_convert_type(v, jnp.uint32)
    e = (bits >> np.uint32(23)).astype(jnp.int32) - 127
    m = lax.bitcast_convert_type(
        (bits & np.uint32(0x7FFFFF)) | np.uint32(0x3F800000), jnp.float32)
    big = m >= _SQRT2
    e = jnp.where(big, e + 1, e).astype(jnp.float32)
    m = jnp.where(big, m * np.float32(0.5), m)
    t = m - np.float32(1.0)
    q = _QC[0]
    for c in _QC[1:]:
        q = q * t + c
    return e * _LN2 + (t + t * t * q)


@functools.partial(
    pl.kernel,
    mesh=plsc.VectorSubcoreMesh(core_axis_name="c", subcore_axis_name="s"),
    compiler_params=pltpu.CompilerParams(needs_layout_passes=False),
    out_type=[
        jax.ShapeDtypeStruct((ROWS, 16), jnp.float32),
        jax.ShapeDtypeStruct((ROWS, 16), jnp.int32),
    ],
    scratch_types=[
        pltpu.VMEM((SC_BLK,), jnp.float32),
        pltpu.VMEM((16,), jnp.float32),
        pltpu.VMEM((16,), jnp.int32),
    ],
)
def _sc_kernel(x_hbm, vals_hbm, idxs_hbm, xb, resv, resi):
    wid = lax.axis_index("s") * 2 + lax.axis_index("c")
    lanes = lax.iota(jnp.int32, 16)

    def scan_row(row):
        rowbase = row * VOCAB

        def inner(j, carry, col0):
            bestv, besti = carry
            xv = xb[pl.ds(j * 16, 16)]
            col = col0 + j * 16 + lanes
            flat = (rowbase + col).astype(jnp.uint32)
            u = _bits_to_u(_threefry_bits(flat))
            g = -_ln16(-_ln16(u))
            y = xv + g
            pred = y > bestv
            return jnp.where(pred, y, bestv), jnp.where(pred, col, besti)

        def block(b, carry):
            col0 = TC_COLS + b * SC_BLK
            pltpu.sync_copy(x_hbm.at[row, pl.ds(col0, SC_BLK)], xb)
            return lax.fori_loop(0, SC_BLK // 16,
                                 lambda j, c: inner(j, c, col0), carry)

        carry = (jnp.full((16,), -jnp.inf, jnp.float32),
                 jnp.zeros((16,), jnp.int32))
        bestv, besti = lax.fori_loop(0, SC_NBLK, block, carry)
        resv[...] = bestv
        resi[...] = besti
        pltpu.sync_copy(resv, vals_hbm.at[row])
        pltpu.sync_copy(resi, idxs_hbm.at[row])

    scan_row(wid * 2)
    scan_row(wid * 2 + 1)


def _merge_body(tcv_ref, tci_ref, scv_ref, sci_ref, out_ref):
    scv = scv_ref[...]
    m = jnp.max(scv, axis=1, keepdims=True)
    si = jnp.min(jnp.where(scv == m, sci_ref[...], jnp.int32(2**31 - 1)),
                 axis=1, keepdims=True)
    out_ref[...] = jnp.where(m > tcv_ref[...], si, tci_ref[...])


def kernel(x):
    tcv, tci = pl.pallas_call(
        _tc_body,
        grid=(NT,),
        in_specs=[pl.BlockSpec((ROWS, CHUNK),
                               lambda i: (0, jnp.where(i < NT1, i, TAILBLK)))],
        out_specs=[pl.BlockSpec((ROWS, 1), lambda i: (0, 0)),
                   pl.BlockSpec((ROWS, 1), lambda i: (0, 0))],
        out_shape=[jax.ShapeDtypeStruct((ROWS, 1), jnp.float32),
                   jax.ShapeDtypeStruct((ROWS, 1), jnp.int32)],
    )(x)
    scv, sci = _sc_kernel(x)
    out = pl.pallas_call(
        _merge_body,
        out_shape=jax.ShapeDtypeStruct((ROWS, 1), jnp.int32),
    )(tcv, tci, scv, sci)
    return out.reshape(ROWS)
